# baseline (speedup 1.0000x reference)
import dataclasses
import functools

import jax
import jax.numpy as jnp
from jax import lax
from jax.experimental import pallas as pl
from jax.experimental.pallas import tpu as pltpu
from jax.experimental.pallas import tpu_sc as plsc

N = 10000
E = 160000
IN_DIM = 256
HID = 64
OUT_DIM = 256

_BLK = 1000
_K = 64
_NCH = 160
_CG = 8
_NPAD = N + 16
_NTILES = 16
_RPT_A = 632
_RPT_B = N - 15 * _RPT_A


def _pre_outs(chalf, h, hst_ref, al_src_ref, al_dst_ref, ni_ref, di_ref,
              a_src, a_dst):
    hst_ref[0] = h[:, :chalf]
    hst_ref[1] = h[:, chalf:]
    al_s = h @ a_src
    al_d = h @ a_dst
    al_src_ref[...] = jnp.broadcast_to(al_s[:, None], al_src_ref.shape)
    al_dst_ref[...] = jnp.broadcast_to(al_d[:, None], al_dst_ref.shape)
    e = al_s + al_d
    w_self = jnp.exp(jnp.maximum(e, 0.2 * e))
    ni = w_self[:, None] * h
    ni_ref[0] = ni[:, :chalf]
    ni_ref[1] = ni[:, chalf:]
    di_ref[...] = jnp.broadcast_to(w_self[:, None], di_ref.shape)


def _pre_out_specs(m, ch):
    return (
        [
            pl.BlockSpec((2, _BLK, ch), lambda i: (0, i, 0)),
            pl.BlockSpec((_BLK, 16), lambda i: (i, 0)),
            pl.BlockSpec((_BLK, 16), lambda i: (i, 0)),
            pl.BlockSpec((2, _BLK, ch), lambda i: (0, i, 0)),
            pl.BlockSpec((_BLK, 16), lambda i: (i, 0)),
        ],
        [
            jax.ShapeDtypeStruct((2, m, ch), jnp.float32),
            jax.ShapeDtypeStruct((m, 16), jnp.float32),
            jax.ShapeDtypeStruct((m, 16), jnp.float32),
            jax.ShapeDtypeStruct((2, m, ch), jnp.float32),
            jax.ShapeDtypeStruct((m, 16), jnp.float32),
        ],
    )


def _ego_body(x_ref, w1_ref, b1_ref, w2_ref, b2_ref, o_ref):
    h = jnp.dot(x_ref[...], w1_ref[...], preferred_element_type=jnp.float32)
    h = h + b1_ref[...]
    o = jnp.dot(h, w2_ref[...], preferred_element_type=jnp.float32)
    o_ref[...] = o + b2_ref[...]


def _ego_mlp(x, W1, b1, W2, b2):
    m, k = x.shape
    h = W1.shape[1]
    n = W2.shape[1]
    return pl.pallas_call(
        _ego_body,
        grid=(m // _BLK,),
        in_specs=[
            pl.BlockSpec((_BLK, k), lambda i: (i, 0)),
            pl.BlockSpec((k, h), lambda i: (0, 0)),
            pl.BlockSpec((h,), lambda i: (0,)),
            pl.BlockSpec((h, n), lambda i: (0, 0)),
            pl.BlockSpec((n,), lambda i: (0,)),
        ],
        out_specs=pl.BlockSpec((_BLK, n), lambda i: (i, 0)),
        out_shape=jax.ShapeDtypeStruct((m, n), jnp.float32),
    )(x, W1, b1, W2, b2)


def _pre_body(chalf, x_ref, w_ref, asrc_ref, adst_ref, hst_ref, al_src_ref,
              al_dst_ref, ni_ref, di_ref):
    h = jnp.dot(x_ref[...], w_ref[...], preferred_element_type=jnp.float32)
    _pre_outs(chalf, h, hst_ref, al_src_ref, al_dst_ref, ni_ref, di_ref,
              asrc_ref[...], adst_ref[...])


def _gat_pre(x, W, a_src, a_dst):
    m, k = x.shape
    c = W.shape[1]
    ch = c // 2
    out_specs, out_shape = _pre_out_specs(m, ch)
    return pl.pallas_call(
        functools.partial(_pre_body, ch),
        grid=(m // _BLK,),
        in_specs=[
            pl.BlockSpec((_BLK, k), lambda i: (i, 0)),
            pl.BlockSpec((k, c), lambda i: (0, 0)),
            pl.BlockSpec((c,), lambda i: (0,)),
            pl.BlockSpec((c,), lambda i: (0,)),
        ],
        out_specs=out_specs,
        out_shape=out_shape,
    )(x, W, a_src, a_dst)


def _post_pre_body(chin, chalf, num_ref, den_ref, di_ref, b_ref, w_ref,
                   asrc_ref, adst_ref, hst_ref, al_src_ref, al_dst_ref,
                   ni_ref, di2_ref):
    num = jnp.concatenate([num_ref[0], num_ref[1]], axis=1)
    den = den_ref[0] + den_ref[1] - di_ref[...]
    out1 = num / (den[:, 0:1] + 1e-16) + b_ref[...]
    h = jnp.dot(out1, w_ref[...], preferred_element_type=jnp.float32)
    _pre_outs(chalf, h, hst_ref, al_src_ref, al_dst_ref, ni_ref, di2_ref,
              asrc_ref[...], adst_ref[...])


def _post1_and_pre2(num_st, den2, di, b, W, a_src, a_dst):
    _, m, chin = num_st.shape
    cin = 2 * chin
    c = W.shape[1]
    ch = c // 2
    out_specs, out_shape = _pre_out_specs(m, ch)
    return pl.pallas_call(
        functools.partial(_post_pre_body, chin, ch),
        grid=(m // _BLK,),
        in_specs=[
            pl.BlockSpec((2, _BLK, chin), lambda i: (0, i, 0)),
            pl.BlockSpec((2, _BLK, 16), lambda i: (0, i, 0)),
            pl.BlockSpec((_BLK, 16), lambda i: (i, 0)),
            pl.BlockSpec((cin,), lambda i: (0,)),
            pl.BlockSpec((cin, c), lambda i: (0, 0)),
            pl.BlockSpec((c,), lambda i: (0,)),
            pl.BlockSpec((c,), lambda i: (0,)),
        ],
        out_specs=out_specs,
        out_shape=out_shape,
    )(num_st, den2, di, b, W, a_src, a_dst)


def _post_body(num_ref, den_ref, di_ref, b_ref, o_ref):
    num = jnp.concatenate([num_ref[0], num_ref[1]], axis=1)
    den = den_ref[0] + den_ref[1] - di_ref[...]
    o_ref[...] = num / (den[:, 0:1] + 1e-16) + b_ref[...]


def _gat_post(num_st, den2, di, b):
    _, m, ch = num_st.shape
    c = 2 * ch
    return pl.pallas_call(
        _post_body,
        grid=(m // _BLK,),
        in_specs=[
            pl.BlockSpec((2, _BLK, ch), lambda i: (0, i, 0)),
            pl.BlockSpec((2, _BLK, 16), lambda i: (0, i, 0)),
            pl.BlockSpec((_BLK, 16), lambda i: (i, 0)),
            pl.BlockSpec((c,), lambda i: (0,)),
        ],
        out_specs=pl.BlockSpec((_BLK, c), lambda i: (i, 0)),
        out_shape=jax.ShapeDtypeStruct((m, c), jnp.float32),
    )(num_st, den2, di, b)


def _sc_edge_body(chalf, hst_hbm, asrc_hbm, adst_hbm, ni_hbm, di_hbm,
                  srcm_hbm, dstm_hbm, num_out, den_out,
                  asrc_t, adst_t, src_t, dst_t, dsave,
                  rows0, rows1, denr0, denr1, w0, w1,
                  num_sh, den_sh, gsem0, gsem1, ssem0, ssem1):
    cid = lax.axis_index("c")
    sid = lax.axis_index("s")
    base = sid * _RPT_A
    coff = cid * N

    pltpu.sync_copy(asrc_hbm, asrc_t)
    pltpu.sync_copy(adst_hbm, adst_t)
    pltpu.sync_copy(srcm_hbm.at[sid, pl.ds(0, _CG)], src_t)
    pltpu.sync_copy(dstm_hbm.at[sid, pl.ds(0, _CG)], dst_t)

    @pl.when(sid < _NTILES - 1)
    def _():
        pltpu.sync_copy(ni_hbm.at[cid, pl.ds(base, _RPT_A)],
                        num_sh.at[pl.ds(base, _RPT_A)])
        pltpu.sync_copy(di_hbm.at[pl.ds(base, _RPT_A)],
                        den_sh.at[pl.ds(base, _RPT_A)])

    @pl.when(sid == _NTILES - 1)
    def _():
        pltpu.sync_copy(ni_hbm.at[cid, pl.ds(base, _RPT_B)],
                        num_sh.at[pl.ds(base, _RPT_B)])
        pltpu.sync_copy(di_hbm.at[pl.ds(base, _RPT_B)],
                        den_sh.at[pl.ds(base, _RPT_B)])

    plsc.subcore_barrier()

    def _wgrp(j, wbuf, dpar):
        srow = src_t.at[j]
        drow = dst_t.at[j]

        @pl.loop(0, _K, step=16, unroll=True)
        def _(g):
            s16 = srow[pl.ds(g, 16)]
            d16 = drow[pl.ds(g, 16)]
            av = plsc.load_gather(asrc_t, [s16])
            bv = plsc.load_gather(adst_t, [d16])
            e = av + bv
            e = jnp.maximum(e, 0.2 * e)
            wbuf[pl.ds(g, 16)] = jnp.exp(e)
            srow[pl.ds(g, 16)] = s16 + coff
            dsave[dpar, pl.ds(g, 16)] = d16

    def _scale(rowsb, wbuf, denb, fill_den):
        @plsc.parallel_loop(0, _K, unroll=8)
        def _(i):
            iv = lax.broadcast(i, (16,))
            wv = plsc.load_gather(wbuf, [iv])
            if fill_den:
                denb[i, :] = wv
            for cc in range(chalf // 16):
                sl = (i, pl.ds(cc * 16, 16))
                rowsb[sl] = rowsb[sl] * wv

    def _half(c, parx, rowsx, denrx, wx, gsemx, ssemx,
              rowsy, denry, wy, gsemy, ssemy):
        pary = 1 - parx

        @pl.when(c > 0)
        def _():
            pltpu.make_async_copy(hst_hbm.at[pl.ds(0, _K)], rowsy, ssemy).wait()

            @pl.when(cid == pary)
            def _():
                pltpu.make_async_copy(di_hbm.at[pl.ds(0, _K)], denry,
                                      ssemy).wait()

        nxt = c + 1
        bnd = jnp.logical_and(nxt % _CG == 0, nxt < _NCH)

        @pl.when(bnd)
        def _():
            pltpu.make_async_copy(hst_hbm.at[pl.ds(0, _K)], rowsx, gsemx).wait()
            pltpu.sync_copy(srcm_hbm.at[sid, pl.ds(nxt, _CG)], src_t)
            pltpu.sync_copy(dstm_hbm.at[sid, pl.ds(nxt, _CG)], dst_t)

        @pl.when(nxt < _NCH)
        def _():
            jn = lax.rem(nxt, _CG)
            _wgrp(jn, wy, pary)
            pltpu.async_copy(hst_hbm.at[src_t.at[jn]], rowsy, gsemy)

        @pl.when(jnp.logical_not(bnd))
        def _():
            pltpu.make_async_copy(hst_hbm.at[pl.ds(0, _K)], rowsx, gsemx).wait()

        @pl.when(cid == parx)
        def _():
            _scale(rowsx, wx, denrx, True)

        @pl.when(cid != parx)
        def _():
            _scale(rowsx, wx, denrx, False)

        pltpu.async_copy(rowsx, num_sh.at[dsave.at[parx]], ssemx, add=True)

        @pl.when(cid == parx)
        def _():
            pltpu.async_copy(denrx, den_sh.at[dsave.at[parx]], ssemx, add=True)

    _wgrp(0, w0, 0)
    pltpu.async_copy(hst_hbm.at[src_t.at[0]], rows0, gsem0)

    @pl.loop(0, _NCH, step=2)
    def _pair(c):
        _half(c, 0, rows0, denr0, w0, gsem0, ssem0,
              rows1, denr1, w1, gsem1, ssem1)
        _half(c + 1, 1, rows1, denr1, w1, gsem1, ssem1,
              rows0, denr0, w0, gsem0, ssem0)

    pltpu.make_async_copy(hst_hbm.at[pl.ds(0, _K)], rows1, ssem1).wait()

    @pl.when(cid == 1)
    def _():
        pltpu.make_async_copy(di_hbm.at[pl.ds(0, _K)], denr1, ssem1).wait()

    plsc.subcore_barrier()

    @pl.when(sid < _NTILES - 1)
    def _():
        pltpu.sync_copy(num_sh.at[pl.ds(base, _RPT_A)],
                        num_out.at[cid, pl.ds(base, _RPT_A)])
        pltpu.sync_copy(den_sh.at[pl.ds(base, _RPT_A)],
                        den_out.at[cid, pl.ds(base, _RPT_A)])

    @pl.when(sid == _NTILES - 1)
    def _():
        pltpu.sync_copy(num_sh.at[pl.ds(base, _RPT_B)],
                        num_out.at[cid, pl.ds(base, _RPT_B)])
        pltpu.sync_copy(den_sh.at[pl.ds(base, _RPT_B)],
                        den_out.at[cid, pl.ds(base, _RPT_B)])


def _sc_edge(hst, asrc, adst, ni_st, di, srcm, dstm, chalf):
    mesh = plsc.VectorSubcoreMesh(core_axis_name="c", subcore_axis_name="s")
    cp = pltpu.CompilerParams()
    if "needs_layout_passes" in pltpu.CompilerParams.__dataclass_fields__:
        cp = dataclasses.replace(cp, needs_layout_passes=False)
    if "use_tc_tiling_on_sc" in pltpu.CompilerParams.__dataclass_fields__:
        cp = dataclasses.replace(cp, use_tc_tiling_on_sc=False)
    kern = pl.kernel(
        functools.partial(_sc_edge_body, chalf),
        mesh=mesh,
        compiler_params=cp,
        out_type=[
            jax.ShapeDtypeStruct((2, N, chalf), jnp.float32),
            jax.ShapeDtypeStruct((2, N, 16), jnp.float32),
        ],
        scratch_types=[
            pltpu.VMEM((_NPAD,), jnp.float32),
            pltpu.VMEM((_NPAD,), jnp.float32),
            pltpu.VMEM((_CG, _K), jnp.int32),
            pltpu.VMEM((_CG, _K), jnp.int32),
            pltpu.VMEM((2, _K), jnp.int32),
            pltpu.VMEM((_K, chalf), jnp.float32),
            pltpu.VMEM((_K, chalf), jnp.float32),
            pltpu.VMEM((_K, 16), jnp.float32),
            pltpu.VMEM((_K, 16), jnp.float32),
            pltpu.VMEM((_K,), jnp.float32),
            pltpu.VMEM((_K,), jnp.float32),
            pltpu.VMEM_SHARED((_NPAD, chalf), jnp.float32),
            pltpu.VMEM_SHARED((_NPAD, 16), jnp.float32),
            pltpu.SemaphoreType.DMA,
            pltpu.SemaphoreType.DMA,
            pltpu.SemaphoreType.DMA,
            pltpu.SemaphoreType.DMA,
        ],
    )
    return kern(hst, asrc, adst, ni_st, di, srcm, dstm)


def _run_sc(hst, al_s, al_d, ni_st, di, srcm, dstm, chalf):
    asrc_flat = jnp.pad(al_s[:, 0], (0, _NPAD - N))
    adst_flat = jnp.pad(al_d[:, 0], (0, _NPAD - N))
    return _sc_edge(hst.reshape(2 * N, chalf), asrc_flat, adst_flat,
                    ni_st, di, srcm, dstm, chalf)


def kernel(x, edge_index, Wego1, bego1, Wego2, bego2, W1, a_src1, a_dst1, b1,
           W2, a_src2, a_dst2, b2):
    pad = _NTILES * _NCH * _K - E
    srcm = jnp.concatenate(
        [edge_index[0], jnp.zeros((pad,), jnp.int32)]).reshape(_NTILES, _NCH, _K)
    dstm = jnp.concatenate(
        [edge_index[1], jnp.full((pad,), N, jnp.int32)]).reshape(_NTILES, _NCH, _K)

    source = _ego_mlp(x, Wego1, bego1, Wego2, bego2)
    hst1, al_s1, al_d1, ni1, di1 = _gat_pre(x, W1, a_src1, a_dst1)
    num1, den1 = _run_sc(hst1, al_s1, al_d1, ni1, di1, srcm, dstm,
                         W1.shape[1] // 2)
    hst2, al_s2, al_d2, ni2, di2 = _post1_and_pre2(
        num1, den1, di1, b1, W2, a_src2, a_dst2)
    num2, den2 = _run_sc(hst2, al_s2, al_d2, ni2, di2, srcm, dstm,
                         W2.shape[1] // 2)
    out2 = _gat_post(num2, den2, di2, b2)
    return (source, out2)


# speedup vs baseline: 1.0593x; 1.0593x over previous
import dataclasses
import functools

import jax
import jax.numpy as jnp
from jax import lax
from jax.experimental import pallas as pl
from jax.experimental.pallas import tpu as pltpu
from jax.experimental.pallas import tpu_sc as plsc

N = 10000
E = 160000
IN_DIM = 256
HID = 64
OUT_DIM = 256

_BLK = 1000
_EPAD = 163840
_K1, _NCH1, _CG1 = 128, 80, 8
_K2, _NCH2, _CG2 = 64, 160, 16
_NPAD = N + 16
_NTILES = 16
_RPT_A = 632
_RPT_B = N - 15 * _RPT_A


def _pre_outs(chalf, h, hst_ref, al_src_ref, al_dst_ref, ni_ref, di_ref,
              a_src, a_dst):
    hst_ref[0] = h[:, :chalf]
    hst_ref[1] = h[:, chalf:]
    al_s = h @ a_src
    al_d = h @ a_dst
    al_src_ref[...] = jnp.broadcast_to(al_s[:, None], al_src_ref.shape)
    al_dst_ref[...] = jnp.broadcast_to(al_d[:, None], al_dst_ref.shape)
    e = al_s + al_d
    w_self = jnp.exp(jnp.maximum(e, 0.2 * e))
    ni = w_self[:, None] * h
    ni_ref[0] = ni[:, :chalf]
    ni_ref[1] = ni[:, chalf:]
    di_ref[...] = jnp.broadcast_to(w_self[:, None], di_ref.shape)


def _pre_out_specs(m, ch):
    return (
        [
            pl.BlockSpec((2, _BLK, ch), lambda i: (0, i, 0)),
            pl.BlockSpec((_BLK, 16), lambda i: (i, 0)),
            pl.BlockSpec((_BLK, 16), lambda i: (i, 0)),
            pl.BlockSpec((2, _BLK, ch), lambda i: (0, i, 0)),
            pl.BlockSpec((_BLK, 16), lambda i: (i, 0)),
        ],
        [
            jax.ShapeDtypeStruct((2, m, ch), jnp.float32),
            jax.ShapeDtypeStruct((m, 16), jnp.float32),
            jax.ShapeDtypeStruct((m, 16), jnp.float32),
            jax.ShapeDtypeStruct((2, m, ch), jnp.float32),
            jax.ShapeDtypeStruct((m, 16), jnp.float32),
        ],
    )


def _ego_body(x_ref, w1_ref, b1_ref, w2_ref, b2_ref, o_ref):
    h = jnp.dot(x_ref[...], w1_ref[...], preferred_element_type=jnp.float32)
    h = h + b1_ref[...]
    o = jnp.dot(h, w2_ref[...], preferred_element_type=jnp.float32)
    o_ref[...] = o + b2_ref[...]


def _ego_mlp(x, W1, b1, W2, b2):
    m, k = x.shape
    h = W1.shape[1]
    n = W2.shape[1]
    return pl.pallas_call(
        _ego_body,
        grid=(m // _BLK,),
        in_specs=[
            pl.BlockSpec((_BLK, k), lambda i: (i, 0)),
            pl.BlockSpec((k, h), lambda i: (0, 0)),
            pl.BlockSpec((h,), lambda i: (0,)),
            pl.BlockSpec((h, n), lambda i: (0, 0)),
            pl.BlockSpec((n,), lambda i: (0,)),
        ],
        out_specs=pl.BlockSpec((_BLK, n), lambda i: (i, 0)),
        out_shape=jax.ShapeDtypeStruct((m, n), jnp.float32),
    )(x, W1, b1, W2, b2)


def _pre_body(chalf, x_ref, w_ref, asrc_ref, adst_ref, hst_ref, al_src_ref,
              al_dst_ref, ni_ref, di_ref):
    h = jnp.dot(x_ref[...], w_ref[...], preferred_element_type=jnp.float32)
    _pre_outs(chalf, h, hst_ref, al_src_ref, al_dst_ref, ni_ref, di_ref,
              asrc_ref[...], adst_ref[...])


def _gat_pre(x, W, a_src, a_dst):
    m, k = x.shape
    c = W.shape[1]
    ch = c // 2
    out_specs, out_shape = _pre_out_specs(m, ch)
    return pl.pallas_call(
        functools.partial(_pre_body, ch),
        grid=(m // _BLK,),
        in_specs=[
            pl.BlockSpec((_BLK, k), lambda i: (i, 0)),
            pl.BlockSpec((k, c), lambda i: (0, 0)),
            pl.BlockSpec((c,), lambda i: (0,)),
            pl.BlockSpec((c,), lambda i: (0,)),
        ],
        out_specs=out_specs,
        out_shape=out_shape,
    )(x, W, a_src, a_dst)


def _post_pre_body(chin, chalf, num_ref, den_ref, di_ref, b_ref, w_ref,
                   asrc_ref, adst_ref, hst_ref, al_src_ref, al_dst_ref,
                   ni_ref, di2_ref):
    num = jnp.concatenate([num_ref[0], num_ref[1]], axis=1)
    den = den_ref[0] + den_ref[1] - di_ref[...]
    out1 = num / (den[:, 0:1] + 1e-16) + b_ref[...]
    h = jnp.dot(out1, w_ref[...], preferred_element_type=jnp.float32)
    _pre_outs(chalf, h, hst_ref, al_src_ref, al_dst_ref, ni_ref, di2_ref,
              asrc_ref[...], adst_ref[...])


def _post1_and_pre2(num_st, den2, di, b, W, a_src, a_dst):
    _, m, chin = num_st.shape
    cin = 2 * chin
    c = W.shape[1]
    ch = c // 2
    out_specs, out_shape = _pre_out_specs(m, ch)
    return pl.pallas_call(
        functools.partial(_post_pre_body, chin, ch),
        grid=(m // _BLK,),
        in_specs=[
            pl.BlockSpec((2, _BLK, chin), lambda i: (0, i, 0)),
            pl.BlockSpec((2, _BLK, 16), lambda i: (0, i, 0)),
            pl.BlockSpec((_BLK, 16), lambda i: (i, 0)),
            pl.BlockSpec((cin,), lambda i: (0,)),
            pl.BlockSpec((cin, c), lambda i: (0, 0)),
            pl.BlockSpec((c,), lambda i: (0,)),
            pl.BlockSpec((c,), lambda i: (0,)),
        ],
        out_specs=out_specs,
        out_shape=out_shape,
    )(num_st, den2, di, b, W, a_src, a_dst)


def _post_body(num_ref, den_ref, di_ref, b_ref, o_ref):
    num = jnp.concatenate([num_ref[0], num_ref[1]], axis=1)
    den = den_ref[0] + den_ref[1] - di_ref[...]
    o_ref[...] = num / (den[:, 0:1] + 1e-16) + b_ref[...]


def _gat_post(num_st, den2, di, b):
    _, m, ch = num_st.shape
    c = 2 * ch
    return pl.pallas_call(
        _post_body,
        grid=(m // _BLK,),
        in_specs=[
            pl.BlockSpec((2, _BLK, ch), lambda i: (0, i, 0)),
            pl.BlockSpec((2, _BLK, 16), lambda i: (0, i, 0)),
            pl.BlockSpec((_BLK, 16), lambda i: (i, 0)),
            pl.BlockSpec((c,), lambda i: (0,)),
        ],
        out_specs=pl.BlockSpec((_BLK, c), lambda i: (i, 0)),
        out_shape=jax.ShapeDtypeStruct((m, c), jnp.float32),
    )(num_st, den2, di, b)


def _sc_edge_body(chalf, _K, _NCH, _CG, hst_hbm, asrc_hbm, adst_hbm, ni_hbm, di_hbm,
                  srcm_hbm, dstm_hbm, num_out, den_out,
                  asrc_t, adst_t, src_t, dst_t, dsave,
                  rows0, rows1, denr0, denr1, w0, w1,
                  num_sh, den_sh, gsem0, gsem1, ssem0, ssem1):
    cid = lax.axis_index("c")
    sid = lax.axis_index("s")
    base = sid * _RPT_A
    coff = cid * N

    pltpu.sync_copy(asrc_hbm, asrc_t)
    pltpu.sync_copy(adst_hbm, adst_t)
    pltpu.sync_copy(srcm_hbm.at[sid, pl.ds(0, _CG)], src_t)
    pltpu.sync_copy(dstm_hbm.at[sid, pl.ds(0, _CG)], dst_t)

    @pl.when(sid < _NTILES - 1)
    def _():
        pltpu.sync_copy(ni_hbm.at[cid, pl.ds(base, _RPT_A)],
                        num_sh.at[pl.ds(base, _RPT_A)])
        pltpu.sync_copy(di_hbm.at[pl.ds(base, _RPT_A)],
                        den_sh.at[pl.ds(base, _RPT_A)])

    @pl.when(sid == _NTILES - 1)
    def _():
        pltpu.sync_copy(ni_hbm.at[cid, pl.ds(base, _RPT_B)],
                        num_sh.at[pl.ds(base, _RPT_B)])
        pltpu.sync_copy(di_hbm.at[pl.ds(base, _RPT_B)],
                        den_sh.at[pl.ds(base, _RPT_B)])

    plsc.subcore_barrier()

    def _wgrp(j, wbuf, dpar):
        srow = src_t.at[j]
        drow = dst_t.at[j]

        @pl.loop(0, _K, step=16, unroll=True)
        def _(g):
            s16 = srow[pl.ds(g, 16)]
            d16 = drow[pl.ds(g, 16)]
            av = plsc.load_gather(asrc_t, [s16])
            bv = plsc.load_gather(adst_t, [d16])
            e = av + bv
            e = jnp.maximum(e, 0.2 * e)
            wbuf[pl.ds(g, 16)] = jnp.exp(e)
            srow[pl.ds(g, 16)] = s16 + coff
            dsave[dpar, pl.ds(g, 16)] = d16

    def _scale(rowsb, wbuf, denb, fill_den):
        @plsc.parallel_loop(0, _K, unroll=8)
        def _(i):
            iv = lax.broadcast(i, (16,))
            wv = plsc.load_gather(wbuf, [iv])
            if fill_den:
                denb[i, :] = wv
            for cc in range(chalf // 16):
                sl = (i, pl.ds(cc * 16, 16))
                rowsb[sl] = rowsb[sl] * wv

    def _half(c, parx, rowsx, denrx, wx, gsemx, ssemx,
              rowsy, denry, wy, gsemy, ssemy):
        pary = 1 - parx

        @pl.when(c > 0)
        def _():
            pltpu.make_async_copy(hst_hbm.at[pl.ds(0, _K)], rowsy, ssemy).wait()

            @pl.when(cid == pary)
            def _():
                pltpu.make_async_copy(di_hbm.at[pl.ds(0, _K)], denry,
                                      ssemy).wait()

        nxt = c + 1
        bnd = jnp.logical_and(nxt % _CG == 0, nxt < _NCH)

        @pl.when(bnd)
        def _():
            pltpu.make_async_copy(hst_hbm.at[pl.ds(0, _K)], rowsx, gsemx).wait()
            pltpu.sync_copy(srcm_hbm.at[sid, pl.ds(nxt, _CG)], src_t)
            pltpu.sync_copy(dstm_hbm.at[sid, pl.ds(nxt, _CG)], dst_t)

        @pl.when(nxt < _NCH)
        def _():
            jn = lax.rem(nxt, _CG)
            _wgrp(jn, wy, pary)
            pltpu.async_copy(hst_hbm.at[src_t.at[jn]], rowsy, gsemy)

        @pl.when(jnp.logical_not(bnd))
        def _():
            pltpu.make_async_copy(hst_hbm.at[pl.ds(0, _K)], rowsx, gsemx).wait()

        @pl.when(cid == parx)
        def _():
            _scale(rowsx, wx, denrx, True)

        @pl.when(cid != parx)
        def _():
            _scale(rowsx, wx, denrx, False)

        pltpu.async_copy(rowsx, num_sh.at[dsave.at[parx]], ssemx, add=True)

        @pl.when(cid == parx)
        def _():
            pltpu.async_copy(denrx, den_sh.at[dsave.at[parx]], ssemx, add=True)

    _wgrp(0, w0, 0)
    pltpu.async_copy(hst_hbm.at[src_t.at[0]], rows0, gsem0)

    @pl.loop(0, _NCH, step=2)
    def _pair(c):
        _half(c, 0, rows0, denr0, w0, gsem0, ssem0,
              rows1, denr1, w1, gsem1, ssem1)
        _half(c + 1, 1, rows1, denr1, w1, gsem1, ssem1,
              rows0, denr0, w0, gsem0, ssem0)

    pltpu.make_async_copy(hst_hbm.at[pl.ds(0, _K)], rows1, ssem1).wait()

    @pl.when(cid == 1)
    def _():
        pltpu.make_async_copy(di_hbm.at[pl.ds(0, _K)], denr1, ssem1).wait()

    plsc.subcore_barrier()

    @pl.when(sid < _NTILES - 1)
    def _():
        pltpu.sync_copy(num_sh.at[pl.ds(base, _RPT_A)],
                        num_out.at[cid, pl.ds(base, _RPT_A)])
        pltpu.sync_copy(den_sh.at[pl.ds(base, _RPT_A)],
                        den_out.at[cid, pl.ds(base, _RPT_A)])

    @pl.when(sid == _NTILES - 1)
    def _():
        pltpu.sync_copy(num_sh.at[pl.ds(base, _RPT_B)],
                        num_out.at[cid, pl.ds(base, _RPT_B)])
        pltpu.sync_copy(den_sh.at[pl.ds(base, _RPT_B)],
                        den_out.at[cid, pl.ds(base, _RPT_B)])


def _sc_edge(hst, asrc, adst, ni_st, di, srcm, dstm, chalf, _K, _NCH, _CG):
    mesh = plsc.VectorSubcoreMesh(core_axis_name="c", subcore_axis_name="s")
    cp = pltpu.CompilerParams()
    if "needs_layout_passes" in pltpu.CompilerParams.__dataclass_fields__:
        cp = dataclasses.replace(cp, needs_layout_passes=False)
    if "use_tc_tiling_on_sc" in pltpu.CompilerParams.__dataclass_fields__:
        cp = dataclasses.replace(cp, use_tc_tiling_on_sc=False)
    kern = pl.kernel(
        functools.partial(_sc_edge_body, chalf, _K, _NCH, _CG),
        mesh=mesh,
        compiler_params=cp,
        out_type=[
            jax.ShapeDtypeStruct((2, N, chalf), jnp.float32),
            jax.ShapeDtypeStruct((2, N, 16), jnp.float32),
        ],
        scratch_types=[
            pltpu.VMEM((_NPAD,), jnp.float32),
            pltpu.VMEM((_NPAD,), jnp.float32),
            pltpu.VMEM((_CG, _K), jnp.int32),
            pltpu.VMEM((_CG, _K), jnp.int32),
            pltpu.VMEM((2, _K), jnp.int32),
            pltpu.VMEM((_K, chalf), jnp.float32),
            pltpu.VMEM((_K, chalf), jnp.float32),
            pltpu.VMEM((_K, 16), jnp.float32),
            pltpu.VMEM((_K, 16), jnp.float32),
            pltpu.VMEM((_K,), jnp.float32),
            pltpu.VMEM((_K,), jnp.float32),
            pltpu.VMEM_SHARED((_NPAD, chalf), jnp.float32),
            pltpu.VMEM_SHARED((_NPAD, 16), jnp.float32),
            pltpu.SemaphoreType.DMA,
            pltpu.SemaphoreType.DMA,
            pltpu.SemaphoreType.DMA,
            pltpu.SemaphoreType.DMA,
        ],
    )
    return kern(hst, asrc, adst, ni_st, di, srcm, dstm)


def _run_sc(hst, al_s, al_d, ni_st, di, srcp, dstp, chalf, k, nch, cg):
    asrc_flat = jnp.pad(al_s[:, 0], (0, _NPAD - N))
    adst_flat = jnp.pad(al_d[:, 0], (0, _NPAD - N))
    srcm = srcp.reshape(_NTILES, nch, k)
    dstm = dstp.reshape(_NTILES, nch, k)
    return _sc_edge(hst.reshape(2 * N, chalf), asrc_flat, adst_flat,
                    ni_st, di, srcm, dstm, chalf, k, nch, cg)


def kernel(x, edge_index, Wego1, bego1, Wego2, bego2, W1, a_src1, a_dst1, b1,
           W2, a_src2, a_dst2, b2):
    pad = _EPAD - E
    srcp = jnp.concatenate([edge_index[0], jnp.zeros((pad,), jnp.int32)])
    dstp = jnp.concatenate([edge_index[1], jnp.full((pad,), N, jnp.int32)])

    source = _ego_mlp(x, Wego1, bego1, Wego2, bego2)
    hst1, al_s1, al_d1, ni1, di1 = _gat_pre(x, W1, a_src1, a_dst1)
    num1, den1 = _run_sc(hst1, al_s1, al_d1, ni1, di1, srcp, dstp,
                         W1.shape[1] // 2, _K1, _NCH1, _CG1)
    hst2, al_s2, al_d2, ni2, di2 = _post1_and_pre2(
        num1, den1, di1, b1, W2, a_src2, a_dst2)
    num2, den2 = _run_sc(hst2, al_s2, al_d2, ni2, di2, srcp, dstp,
                         W2.shape[1] // 2, _K2, _NCH2, _CG2)
    out2 = _gat_post(num2, den2, di2, b2)
    return (source, out2)


# speedup vs baseline: 1.0683x; 1.0086x over previous
import dataclasses
import functools

import jax
import jax.numpy as jnp
from jax import lax
from jax.experimental import pallas as pl
from jax.experimental.pallas import tpu as pltpu
from jax.experimental.pallas import tpu_sc as plsc

N = 10000
E = 160000
IN_DIM = 256
HID = 64
OUT_DIM = 256

_BLK = 1000
_EPAD = 163840
_K1, _NCH1, _CG1 = 128, 80, 8
_K2, _NCH2, _CG2 = 64, 160, 16
_NPAD = N + 16
_NTILES = 16
_RPT_A = 632
_RPT_B = N - 15 * _RPT_A


def _pre_outs(chalf, h, hst_ref, al_src_ref, al_dst_ref, ni_ref, di_ref,
              a_src, a_dst):
    hst_ref[0] = h[:, :chalf]
    hst_ref[1] = h[:, chalf:]
    al_s = h @ a_src
    al_d = h @ a_dst
    al_src_ref[...] = jnp.broadcast_to(al_s[:, None], al_src_ref.shape)
    al_dst_ref[...] = jnp.broadcast_to(al_d[:, None], al_dst_ref.shape)
    e = al_s + al_d
    w_self = jnp.exp(jnp.maximum(e, 0.2 * e))
    ni = w_self[:, None] * h
    ni_ref[0] = ni[:, :chalf]
    ni_ref[1] = ni[:, chalf:]
    di_ref[...] = jnp.broadcast_to(w_self[:, None], di_ref.shape)


def _pre_out_specs(m, ch):
    return (
        [
            pl.BlockSpec((2, _BLK, ch), lambda i: (0, i, 0)),
            pl.BlockSpec((_BLK, 16), lambda i: (i, 0)),
            pl.BlockSpec((_BLK, 16), lambda i: (i, 0)),
            pl.BlockSpec((2, _BLK, ch), lambda i: (0, i, 0)),
            pl.BlockSpec((_BLK, 16), lambda i: (i, 0)),
        ],
        [
            jax.ShapeDtypeStruct((2, m, ch), jnp.float32),
            jax.ShapeDtypeStruct((m, 16), jnp.float32),
            jax.ShapeDtypeStruct((m, 16), jnp.float32),
            jax.ShapeDtypeStruct((2, m, ch), jnp.float32),
            jax.ShapeDtypeStruct((m, 16), jnp.float32),
        ],
    )


def _ego_body(x_ref, w1_ref, b1_ref, w2_ref, b2_ref, o_ref):
    h = jnp.dot(x_ref[...], w1_ref[...], preferred_element_type=jnp.float32)
    h = h + b1_ref[...]
    o = jnp.dot(h, w2_ref[...], preferred_element_type=jnp.float32)
    o_ref[...] = o + b2_ref[...]


def _ego_mlp(x, W1, b1, W2, b2):
    m, k = x.shape
    h = W1.shape[1]
    n = W2.shape[1]
    return pl.pallas_call(
        _ego_body,
        grid=(m // _BLK,),
        in_specs=[
            pl.BlockSpec((_BLK, k), lambda i: (i, 0)),
            pl.BlockSpec((k, h), lambda i: (0, 0)),
            pl.BlockSpec((h,), lambda i: (0,)),
            pl.BlockSpec((h, n), lambda i: (0, 0)),
            pl.BlockSpec((n,), lambda i: (0,)),
        ],
        out_specs=pl.BlockSpec((_BLK, n), lambda i: (i, 0)),
        out_shape=jax.ShapeDtypeStruct((m, n), jnp.float32),
    )(x, W1, b1, W2, b2)


def _pre_body(chalf, x_ref, w_ref, asrc_ref, adst_ref, hst_ref, al_src_ref,
              al_dst_ref, ni_ref, di_ref):
    h = jnp.dot(x_ref[...], w_ref[...], preferred_element_type=jnp.float32)
    _pre_outs(chalf, h, hst_ref, al_src_ref, al_dst_ref, ni_ref, di_ref,
              asrc_ref[...], adst_ref[...])


def _gat_pre(x, W, a_src, a_dst):
    m, k = x.shape
    c = W.shape[1]
    ch = c // 2
    out_specs, out_shape = _pre_out_specs(m, ch)
    return pl.pallas_call(
        functools.partial(_pre_body, ch),
        grid=(m // _BLK,),
        in_specs=[
            pl.BlockSpec((_BLK, k), lambda i: (i, 0)),
            pl.BlockSpec((k, c), lambda i: (0, 0)),
            pl.BlockSpec((c,), lambda i: (0,)),
            pl.BlockSpec((c,), lambda i: (0,)),
        ],
        out_specs=out_specs,
        out_shape=out_shape,
    )(x, W, a_src, a_dst)


def _post_pre_body(chin, chalf, num_ref, den_ref, di_ref, b_ref, w_ref,
                   asrc_ref, adst_ref, hst_ref, al_src_ref, al_dst_ref,
                   ni_ref, di2_ref):
    num = jnp.concatenate([num_ref[0], num_ref[1]], axis=1)
    den = den_ref[0] + den_ref[1] - di_ref[...]
    out1 = num / (den[:, 0:1] + 1e-16) + b_ref[...]
    h = jnp.dot(out1, w_ref[...], preferred_element_type=jnp.float32)
    _pre_outs(chalf, h, hst_ref, al_src_ref, al_dst_ref, ni_ref, di2_ref,
              asrc_ref[...], adst_ref[...])


def _post1_and_pre2(num_st, den2, di, b, W, a_src, a_dst):
    _, m, chin = num_st.shape
    cin = 2 * chin
    c = W.shape[1]
    ch = c // 2
    out_specs, out_shape = _pre_out_specs(m, ch)
    return pl.pallas_call(
        functools.partial(_post_pre_body, chin, ch),
        grid=(m // _BLK,),
        in_specs=[
            pl.BlockSpec((2, _BLK, chin), lambda i: (0, i, 0)),
            pl.BlockSpec((2, _BLK, 16), lambda i: (0, i, 0)),
            pl.BlockSpec((_BLK, 16), lambda i: (i, 0)),
            pl.BlockSpec((cin,), lambda i: (0,)),
            pl.BlockSpec((cin, c), lambda i: (0, 0)),
            pl.BlockSpec((c,), lambda i: (0,)),
            pl.BlockSpec((c,), lambda i: (0,)),
        ],
        out_specs=out_specs,
        out_shape=out_shape,
    )(num_st, den2, di, b, W, a_src, a_dst)


def _post_body(num_ref, den_ref, di_ref, b_ref, o_ref):
    num = jnp.concatenate([num_ref[0], num_ref[1]], axis=1)
    den = den_ref[0] + den_ref[1] - di_ref[...]
    o_ref[...] = num / (den[:, 0:1] + 1e-16) + b_ref[...]


def _gat_post(num_st, den2, di, b):
    _, m, ch = num_st.shape
    c = 2 * ch
    return pl.pallas_call(
        _post_body,
        grid=(m // _BLK,),
        in_specs=[
            pl.BlockSpec((2, _BLK, ch), lambda i: (0, i, 0)),
            pl.BlockSpec((2, _BLK, 16), lambda i: (0, i, 0)),
            pl.BlockSpec((_BLK, 16), lambda i: (i, 0)),
            pl.BlockSpec((c,), lambda i: (0,)),
        ],
        out_specs=pl.BlockSpec((_BLK, c), lambda i: (i, 0)),
        out_shape=jax.ShapeDtypeStruct((m, c), jnp.float32),
    )(num_st, den2, di, b)


def _sc_edge_body(chalf, _K, _NCH, _CG, hst_hbm, asrc_hbm, adst_hbm, ni_hbm, di_hbm,
                  srcm_hbm, dstm_hbm, num_out, den_out,
                  asrc_t, adst_t, src_t, dst_t, dsave,
                  rows0, rows1, denr0, denr1, w0, w1,
                  num_sh, den_sh, gsem0, gsem1, ssem0, ssem1):
    cid = lax.axis_index("c")
    sid = lax.axis_index("s")
    base = sid * _RPT_A
    coff = cid * N

    pltpu.sync_copy(asrc_hbm, asrc_t)
    pltpu.sync_copy(adst_hbm, adst_t)
    pltpu.sync_copy(srcm_hbm.at[sid, pl.ds(0, _CG)], src_t)
    pltpu.sync_copy(dstm_hbm.at[sid, pl.ds(0, _CG)], dst_t)

    @pl.when(sid < _NTILES - 1)
    def _():
        pltpu.sync_copy(ni_hbm.at[cid, pl.ds(base, _RPT_A)],
                        num_sh.at[pl.ds(base, _RPT_A)])
        pltpu.sync_copy(di_hbm.at[pl.ds(base, _RPT_A)],
                        den_sh.at[pl.ds(base, _RPT_A)])

    @pl.when(sid == _NTILES - 1)
    def _():
        pltpu.sync_copy(ni_hbm.at[cid, pl.ds(base, _RPT_B)],
                        num_sh.at[pl.ds(base, _RPT_B)])
        pltpu.sync_copy(di_hbm.at[pl.ds(base, _RPT_B)],
                        den_sh.at[pl.ds(base, _RPT_B)])

    plsc.subcore_barrier()

    def _wgrp(j, wbuf, dpar):
        srow = src_t.at[j]
        drow = dst_t.at[j]
        kh = _K // 2

        for g in range(0, _K, 16):
            s16 = srow[pl.ds(g, 16)]
            d16 = drow[pl.ds(g, 16)]
            av = plsc.load_gather(asrc_t, [s16])
            bv = plsc.load_gather(adst_t, [d16])
            e = av + bv
            e = jnp.maximum(e, 0.2 * e)
            wbuf[pl.ds(g, 16)] = jnp.exp(e)
            srow[pl.ds(g, 16)] = s16 + coff
            dsave[dpar, g // kh, pl.ds(g % kh, 16)] = d16

    def _scale(rowsb, wbuf, denb, fill_den, lo, hi):
        @plsc.parallel_loop(lo, hi, unroll=8)
        def _(i):
            iv = lax.broadcast(i, (16,))
            wv = plsc.load_gather(wbuf, [iv])
            if fill_den:
                denb[i, :] = wv
            for cc in range(chalf // 16):
                sl = (i, pl.ds(cc * 16, 16))
                rowsb[sl] = rowsb[sl] * wv

    def _half(c, parx, rowsx, denrx, wx, gsemx, ssemx,
              rowsy, denry, wy, gsemy, ssemy):
        pary = 1 - parx

        @pl.when(c > 0)
        def _():
            pltpu.make_async_copy(hst_hbm.at[pl.ds(0, _K)], rowsy, ssemy).wait()

            @pl.when(cid == pary)
            def _():
                pltpu.make_async_copy(di_hbm.at[pl.ds(0, _K)], denry,
                                      ssemy).wait()

        nxt = c + 1
        bnd = jnp.logical_and(nxt % _CG == 0, nxt < _NCH)

        @pl.when(bnd)
        def _():
            pltpu.make_async_copy(hst_hbm.at[pl.ds(0, _K)], rowsx, gsemx).wait()
            pltpu.sync_copy(srcm_hbm.at[sid, pl.ds(nxt, _CG)], src_t)
            pltpu.sync_copy(dstm_hbm.at[sid, pl.ds(nxt, _CG)], dst_t)

        @pl.when(nxt < _NCH)
        def _():
            jn = lax.rem(nxt, _CG)
            _wgrp(jn, wy, pary)
            pltpu.async_copy(hst_hbm.at[src_t.at[jn]], rowsy, gsemy)

        @pl.when(jnp.logical_not(bnd))
        def _():
            pltpu.make_async_copy(hst_hbm.at[pl.ds(0, _K)], rowsx, gsemx).wait()

        kh = _K // 2
        for hf in (0, 1):
            lo = hf * kh

            @pl.when(cid == parx)
            def _():
                _scale(rowsx, wx, denrx, True, lo, lo + kh)

            @pl.when(cid != parx)
            def _():
                _scale(rowsx, wx, denrx, False, lo, lo + kh)

            pltpu.async_copy(rowsx.at[pl.ds(lo, kh)],
                             num_sh.at[dsave.at[parx, hf]], ssemx, add=True)

            @pl.when(cid == parx)
            def _():
                pltpu.async_copy(denrx.at[pl.ds(lo, kh)],
                                 den_sh.at[dsave.at[parx, hf]], ssemx,
                                 add=True)

    _wgrp(0, w0, 0)
    pltpu.async_copy(hst_hbm.at[src_t.at[0]], rows0, gsem0)

    @pl.loop(0, _NCH, step=2)
    def _pair(c):
        _half(c, 0, rows0, denr0, w0, gsem0, ssem0,
              rows1, denr1, w1, gsem1, ssem1)
        _half(c + 1, 1, rows1, denr1, w1, gsem1, ssem1,
              rows0, denr0, w0, gsem0, ssem0)

    pltpu.make_async_copy(hst_hbm.at[pl.ds(0, _K)], rows1, ssem1).wait()

    @pl.when(cid == 1)
    def _():
        pltpu.make_async_copy(di_hbm.at[pl.ds(0, _K)], denr1, ssem1).wait()

    plsc.subcore_barrier()

    @pl.when(sid < _NTILES - 1)
    def _():
        pltpu.sync_copy(num_sh.at[pl.ds(base, _RPT_A)],
                        num_out.at[cid, pl.ds(base, _RPT_A)])
        pltpu.sync_copy(den_sh.at[pl.ds(base, _RPT_A)],
                        den_out.at[cid, pl.ds(base, _RPT_A)])

    @pl.when(sid == _NTILES - 1)
    def _():
        pltpu.sync_copy(num_sh.at[pl.ds(base, _RPT_B)],
                        num_out.at[cid, pl.ds(base, _RPT_B)])
        pltpu.sync_copy(den_sh.at[pl.ds(base, _RPT_B)],
                        den_out.at[cid, pl.ds(base, _RPT_B)])


def _sc_edge(hst, asrc, adst, ni_st, di, srcm, dstm, chalf, _K, _NCH, _CG):
    mesh = plsc.VectorSubcoreMesh(core_axis_name="c", subcore_axis_name="s")
    cp = pltpu.CompilerParams()
    if "needs_layout_passes" in pltpu.CompilerParams.__dataclass_fields__:
        cp = dataclasses.replace(cp, needs_layout_passes=False)
    if "use_tc_tiling_on_sc" in pltpu.CompilerParams.__dataclass_fields__:
        cp = dataclasses.replace(cp, use_tc_tiling_on_sc=False)
    kern = pl.kernel(
        functools.partial(_sc_edge_body, chalf, _K, _NCH, _CG),
        mesh=mesh,
        compiler_params=cp,
        out_type=[
            jax.ShapeDtypeStruct((2, N, chalf), jnp.float32),
            jax.ShapeDtypeStruct((2, N, 16), jnp.float32),
        ],
        scratch_types=[
            pltpu.VMEM((_NPAD,), jnp.float32),
            pltpu.VMEM((_NPAD,), jnp.float32),
            pltpu.VMEM((_CG, _K), jnp.int32),
            pltpu.VMEM((_CG, _K), jnp.int32),
            pltpu.VMEM((2, 2, _K // 2), jnp.int32),
            pltpu.VMEM((_K, chalf), jnp.float32),
            pltpu.VMEM((_K, chalf), jnp.float32),
            pltpu.VMEM((_K, 16), jnp.float32),
            pltpu.VMEM((_K, 16), jnp.float32),
            pltpu.VMEM((_K,), jnp.float32),
            pltpu.VMEM((_K,), jnp.float32),
            pltpu.VMEM_SHARED((_NPAD, chalf), jnp.float32),
            pltpu.VMEM_SHARED((_NPAD, 16), jnp.float32),
            pltpu.SemaphoreType.DMA,
            pltpu.SemaphoreType.DMA,
            pltpu.SemaphoreType.DMA,
            pltpu.SemaphoreType.DMA,
        ],
    )
    return kern(hst, asrc, adst, ni_st, di, srcm, dstm)


def _run_sc(hst, al_s, al_d, ni_st, di, srcp, dstp, chalf, k, nch, cg):
    asrc_flat = jnp.pad(al_s[:, 0], (0, _NPAD - N))
    adst_flat = jnp.pad(al_d[:, 0], (0, _NPAD - N))
    srcm = srcp.reshape(_NTILES, nch, k)
    dstm = dstp.reshape(_NTILES, nch, k)
    return _sc_edge(hst.reshape(2 * N, chalf), asrc_flat, adst_flat,
                    ni_st, di, srcm, dstm, chalf, k, nch, cg)


def kernel(x, edge_index, Wego1, bego1, Wego2, bego2, W1, a_src1, a_dst1, b1,
           W2, a_src2, a_dst2, b2):
    pad = _EPAD - E
    srcp = jnp.concatenate([edge_index[0], jnp.zeros((pad,), jnp.int32)])
    dstp = jnp.concatenate([edge_index[1], jnp.full((pad,), N, jnp.int32)])

    source = _ego_mlp(x, Wego1, bego1, Wego2, bego2)
    hst1, al_s1, al_d1, ni1, di1 = _gat_pre(x, W1, a_src1, a_dst1)
    num1, den1 = _run_sc(hst1, al_s1, al_d1, ni1, di1, srcp, dstp,
                         W1.shape[1] // 2, _K1, _NCH1, _CG1)
    hst2, al_s2, al_d2, ni2, di2 = _post1_and_pre2(
        num1, den1, di1, b1, W2, a_src2, a_dst2)
    num2, den2 = _run_sc(hst2, al_s2, al_d2, ni2, di2, srcp, dstp,
                         W2.shape[1] // 2, _K2, _NCH2, _CG2)
    out2 = _gat_post(num2, den2, di2, b2)
    return (source, out2)
